# baseline (speedup 1.0000x reference)
import jax
import jax.numpy as jnp
from jax import lax
from jax.experimental import pallas as pl
from jax.experimental.pallas import tpu as pltpu

_N = 1_000_000
_C = 1024
_BLK = 131072
_NSTEP = 8

_K1 = 42
_K2 = (42 ^ 0x1BD11BDA) & 0xFFFFFFFF


def _threefry_bits(x1):
    def rot(x, r):
        return (x << jnp.uint32(r)) | (x >> jnp.uint32(32 - r))

    x0 = x1
    x1 = rot(x1, 13) ^ x0
    for r in (15, 26, 6):
        x0 = x0 + x1
        x1 = rot(x1, r) ^ x0
    x0 = x0 + jnp.uint32(_K1)
    x1 = x1 + jnp.uint32((_K2 + 1) & 0xFFFFFFFF)

    for r in (17, 29, 16, 24):
        x0 = x0 + x1
        x1 = rot(x1, r) ^ x0
    x0 = x0 + jnp.uint32(_K2)
    x1 = x1 + jnp.uint32(2)

    for r in (13, 15, 26, 6):
        x0 = x0 + x1
        x1 = rot(x1, r) ^ x0
    x1 = x1 + jnp.uint32((_K1 + 3) & 0xFFFFFFFF)

    for r in (17, 29, 16, 24):
        x0 = x0 + x1
        x1 = rot(x1, r) ^ x0
    x0 = x0 + jnp.uint32(_K1)
    x1 = x1 + jnp.uint32((_K2 + 4) & 0xFFFFFFFF)

    for r in (13, 15, 26, 6):
        x0 = x0 + x1
        x1 = rot(x1, r) ^ x0
    x0 = x0 + jnp.uint32(_K2)
    x1 = x1 + jnp.uint32(5)

    return x0 ^ x1


def _gumbel_from_x1(x1):
    bits = _threefry_bits(x1)
    fbits = (bits >> jnp.uint32(9)) | jnp.uint32(0x3F800000)
    f = lax.bitcast_convert_type(fbits, jnp.float32)
    eps = jnp.float32(1e-10)
    u = (f - jnp.float32(1.0)) + eps
    return -jnp.log(-jnp.log(u))


def _chunk_plan(step):
    gb = step * _BLK
    if step < _NSTEP - 1:
        return [(k * _C, gb + k * _C) for k in range(_BLK // _C)]
    valid = _N - gb
    nfull = valid // _C
    plan = [(k * _C, gb + k * _C) for k in range(nfull)]
    if valid % _C:
        plan.append((valid - _C, _N - _C))
    return plan


def _body(l_ref, out_ref, zms, ims):
    b = pl.program_id(0)
    row = lax.broadcasted_iota(jnp.int32, (8, 128), 0)
    col = lax.broadcasted_iota(jnp.int32, (8, 128), 1)
    rck = lax.bitcast_convert_type(row * 128 + col + _K1, jnp.uint32)

    def accum(zm, im, local, gbase):
        x1 = rck + jnp.uint32(gbase)
        z = jnp.reshape(l_ref[pl.ds(local, _C)], (8, 128)) + _gumbel_from_x1(x1)
        upd = z > zm
        return (jnp.where(upd, z, zm),
                jnp.where(upd, lax.bitcast_convert_type(x1, jnp.int32), im))

    for step in range(_NSTEP):
        @pl.when(b == step)
        def _(step=step):
            if step == 0:
                zm = jnp.full((8, 128), -jnp.inf, jnp.float32)
                im = jnp.zeros((8, 128), jnp.int32)
            else:
                zm = zms[...]
                im = ims[...]
            for local, gbase in _chunk_plan(step):
                zm, im = accum(zm, im, local, gbase)
            if step < _NSTEP - 1:
                zms[...] = zm
                ims[...] = im
            else:
                m = jnp.max(zm)
                cand = jnp.where(zm == m, im, jnp.int32(0x7FFFFFFF))
                out_ref[0] = jnp.min(cand) - _K1


def kernel(logits):
    out = pl.pallas_call(
        _body,
        grid=(_NSTEP,),
        in_specs=[pl.BlockSpec((_BLK,), lambda i: (i,))],
        out_specs=pl.BlockSpec(memory_space=pltpu.SMEM),
        out_shape=jax.ShapeDtypeStruct((1,), jnp.int32),
        scratch_shapes=[
            pltpu.VMEM((8, 128), jnp.float32),
            pltpu.VMEM((8, 128), jnp.int32),
        ],
    )(logits)
    return out[0]


# speedup vs baseline: 1.0097x; 1.0097x over previous
import jax
import jax.numpy as jnp
from jax import lax
from jax.experimental import pallas as pl
from jax.experimental.pallas import tpu as pltpu

_N = 1_000_000
_C = 1024
_BLK = 262144
_NSTEP = 4

_K1 = 42
_K2 = (42 ^ 0x1BD11BDA) & 0xFFFFFFFF


def _threefry_bits(x1):
    def rot(x, r):
        return (x << jnp.uint32(r)) | (x >> jnp.uint32(32 - r))

    x0 = x1
    x1 = rot(x1, 13) ^ x0
    for r in (15, 26, 6):
        x0 = x0 + x1
        x1 = rot(x1, r) ^ x0
    x0 = x0 + jnp.uint32(_K1)
    x1 = x1 + jnp.uint32((_K2 + 1) & 0xFFFFFFFF)

    for r in (17, 29, 16, 24):
        x0 = x0 + x1
        x1 = rot(x1, r) ^ x0
    x0 = x0 + jnp.uint32(_K2)
    x1 = x1 + jnp.uint32(2)

    for r in (13, 15, 26, 6):
        x0 = x0 + x1
        x1 = rot(x1, r) ^ x0
    x1 = x1 + jnp.uint32((_K1 + 3) & 0xFFFFFFFF)

    for r in (17, 29, 16, 24):
        x0 = x0 + x1
        x1 = rot(x1, r) ^ x0
    x0 = x0 + jnp.uint32(_K1)
    x1 = x1 + jnp.uint32((_K2 + 4) & 0xFFFFFFFF)

    for r in (13, 15, 26, 6):
        x0 = x0 + x1
        x1 = rot(x1, r) ^ x0
    x0 = x0 + jnp.uint32(_K2)
    x1 = x1 + jnp.uint32(5)

    return x0 ^ x1


def _gumbel_from_x1(x1):
    bits = _threefry_bits(x1)
    fbits = (bits >> jnp.uint32(9)) | jnp.uint32(0x3F800000)
    f = lax.bitcast_convert_type(fbits, jnp.float32)
    eps = jnp.float32(1e-10)
    u = (f - jnp.float32(1.0)) + eps
    return -jnp.log(-jnp.log(u))


def _chunk_plan(step):
    gb = step * _BLK
    if step < _NSTEP - 1:
        return [(k * _C, gb + k * _C) for k in range(_BLK // _C)]
    valid = _N - gb
    nfull = valid // _C
    plan = [(k * _C, gb + k * _C) for k in range(nfull)]
    if valid % _C:
        plan.append((valid - _C, _N - _C))
    return plan


def _body(l_ref, out_ref, zms, ims):
    b = pl.program_id(0)
    row = lax.broadcasted_iota(jnp.int32, (8, 128), 0)
    col = lax.broadcasted_iota(jnp.int32, (8, 128), 1)
    rck = lax.bitcast_convert_type(row * 128 + col + _K1, jnp.uint32)

    def accum(zm, im, local, gbase):
        x1 = rck + jnp.uint32(gbase)
        z = jnp.reshape(l_ref[pl.ds(local, _C)], (8, 128)) + _gumbel_from_x1(x1)
        upd = z > zm
        return (jnp.where(upd, z, zm),
                jnp.where(upd, lax.bitcast_convert_type(x1, jnp.int32), im))

    for step in range(_NSTEP):
        @pl.when(b == step)
        def _(step=step):
            if step == 0:
                zm = jnp.full((8, 128), -jnp.inf, jnp.float32)
                im = jnp.zeros((8, 128), jnp.int32)
            else:
                zm = zms[...]
                im = ims[...]
            for local, gbase in _chunk_plan(step):
                zm, im = accum(zm, im, local, gbase)
            if step < _NSTEP - 1:
                zms[...] = zm
                ims[...] = im
            else:
                m = jnp.max(zm)
                cand = jnp.where(zm == m, im, jnp.int32(0x7FFFFFFF))
                out_ref[0] = jnp.min(cand) - _K1


def kernel(logits):
    out = pl.pallas_call(
        _body,
        grid=(_NSTEP,),
        in_specs=[pl.BlockSpec((_BLK,), lambda i: (i,))],
        out_specs=pl.BlockSpec(memory_space=pltpu.SMEM),
        out_shape=jax.ShapeDtypeStruct((1,), jnp.int32),
        scratch_shapes=[
            pltpu.VMEM((8, 128), jnp.float32),
            pltpu.VMEM((8, 128), jnp.int32),
        ],
    )(logits)
    return out[0]


# speedup vs baseline: 1.0161x; 1.0064x over previous
import jax
import jax.numpy as jnp
from jax import lax
from jax.experimental import pallas as pl
from jax.experimental.pallas import tpu as pltpu

_N = 1_000_000
_C = 1024
_BLK = 262144
_NSTEP = 4

_K1 = 42
_K2 = (42 ^ 0x1BD11BDA) & 0xFFFFFFFF


def _threefry_bits(x1):
    def rot(x, r):
        return (x << jnp.uint32(r)) | (x >> jnp.uint32(32 - r))

    x0 = x1
    x1 = rot(x1, 13) ^ x0
    for r in (15, 26, 6):
        x0 = x0 + x1
        x1 = rot(x1, r) ^ x0
    x0 = x0 + jnp.uint32(_K1)
    x1 = x1 + jnp.uint32((_K2 + 1) & 0xFFFFFFFF)

    for r in (17, 29, 16, 24):
        x0 = x0 + x1
        x1 = rot(x1, r) ^ x0
    x0 = x0 + jnp.uint32(_K2)
    x1 = x1 + jnp.uint32(2)

    for r in (13, 15, 26, 6):
        x0 = x0 + x1
        x1 = rot(x1, r) ^ x0
    x1 = x1 + jnp.uint32((_K1 + 3) & 0xFFFFFFFF)

    for r in (17, 29, 16, 24):
        x0 = x0 + x1
        x1 = rot(x1, r) ^ x0
    x0 = x0 + jnp.uint32(_K1)
    x1 = x1 + jnp.uint32((_K2 + 4) & 0xFFFFFFFF)

    for r in (13, 15, 26, 6):
        x0 = x0 + x1
        x1 = rot(x1, r) ^ x0
    x0 = x0 + jnp.uint32(_K2)
    x1 = x1 + jnp.uint32(5)

    return x0 ^ x1


def _gumbel_from_x1(x1):
    bits = _threefry_bits(x1)
    fbits = (bits >> jnp.uint32(9)) | jnp.uint32(0x3F800000)
    f = lax.bitcast_convert_type(fbits, jnp.float32)
    eps = jnp.float32(1e-10)
    u = (f - jnp.float32(1.0)) + eps
    return jnp.log(-jnp.log(u))


def _chunk_plan(step):
    gb = step * _BLK
    if step < _NSTEP - 1:
        return [(k * _C, gb + k * _C) for k in range(_BLK // _C)]
    valid = _N - gb
    nfull = valid // _C
    plan = [(k * _C, gb + k * _C) for k in range(nfull)]
    if valid % _C:
        plan.append((valid - _C, _N - _C))
    return plan


def _body(l_ref, out_ref, zms, ims):
    b = pl.program_id(0)
    row = lax.broadcasted_iota(jnp.int32, (8, 128), 0)
    col = lax.broadcasted_iota(jnp.int32, (8, 128), 1)
    rck = lax.bitcast_convert_type(row * 128 + col + _K1, jnp.uint32)

    def accum(zm, im, local, gbase):
        x1 = rck + jnp.uint32(gbase)
        z = jnp.reshape(l_ref[pl.ds(local, _C)], (8, 128)) - _gumbel_from_x1(x1)
        upd = z > zm
        return (jnp.where(upd, z, zm),
                jnp.where(upd, lax.bitcast_convert_type(x1, jnp.int32), im))

    for step in range(_NSTEP):
        @pl.when(b == step)
        def _(step=step):
            if step == 0:
                zm = jnp.full((8, 128), -jnp.inf, jnp.float32)
                im = jnp.zeros((8, 128), jnp.int32)
            else:
                zm = zms[...]
                im = ims[...]
            for local, gbase in _chunk_plan(step):
                zm, im = accum(zm, im, local, gbase)
            if step < _NSTEP - 1:
                zms[...] = zm
                ims[...] = im
            else:
                m = jnp.max(zm)
                cand = jnp.where(zm == m, im, jnp.int32(0x7FFFFFFF))
                out_ref[0] = jnp.min(cand) - _K1


def kernel(logits):
    out = pl.pallas_call(
        _body,
        grid=(_NSTEP,),
        in_specs=[pl.BlockSpec((_BLK,), lambda i: (i,))],
        out_specs=pl.BlockSpec(memory_space=pltpu.SMEM),
        out_shape=jax.ShapeDtypeStruct((1,), jnp.int32),
        scratch_shapes=[
            pltpu.VMEM((8, 128), jnp.float32),
            pltpu.VMEM((8, 128), jnp.int32),
        ],
    )(logits)
    return out[0]
